# Initial kernel scaffold; baseline (speedup 1.0000x reference)
import jax
import jax.numpy as jnp
from jax.experimental import pallas as pl


def kernel(x, edge_attr, W1_0, W1_1, b1, W2_0, W2_1, b2, Wf, bf, edge_index, batch, y):
    raise NotImplementedError("write your pallas kernel here")


# speedup vs baseline: 14.9567x; 14.9567x over previous
import functools

import jax
import jax.numpy as jnp
from jax import lax
from jax.experimental import pallas as pl
from jax.experimental.pallas import tpu as pltpu
from jax.experimental.pallas import tpu_sc as plsc

_NC, _NS = 2, 16
_NW = _NC * _NS
_NPT = 3136
_NP = _NPT * _NS
_K = 1000


def _mesh():
    return plsc.VectorSubcoreMesh(core_axis_name="c", subcore_axis_name="s")


_SC_PARAMS = pltpu.CompilerParams(needs_layout_passes=False,
                                  use_tc_tiling_on_sc=False)


def _sc_deg(row, edge_attr):
    e_total = edge_attr.shape[0]
    per_w = e_total // _NW

    @functools.partial(
        pl.kernel,
        out_type=jax.ShapeDtypeStruct((_NC * _NP,), jnp.float32),
        mesh=_mesh(),
        scratch_types=[
            pltpu.VMEM((per_w,), jnp.int32),
            pltpu.VMEM((per_w,), jnp.float32),
            pltpu.VMEM((_NPT,), jnp.float32),
            pltpu.VMEM_SHARED((_NP,), jnp.float32),
        ],
        compiler_params=_SC_PARAMS,
    )
    def k(row_hbm, ea_hbm, out_hbm, row_v, ea_v, zb_v, acc_sh):
        c = lax.axis_index("c")
        s = lax.axis_index("s")
        w = c * _NS + s

        def zb(i, carry):
            zb_v[pl.ds(i * 16, 16)] = jnp.zeros((16,), jnp.float32)
            return carry

        lax.fori_loop(0, _NPT // 16, zb, 0)
        pltpu.sync_copy(zb_v, acc_sh.at[pl.ds(s * _NPT, _NPT)])
        plsc.subcore_barrier()

        base = w * per_w
        pltpu.sync_copy(row_hbm.at[pl.ds(base, per_w)], row_v)
        pltpu.sync_copy(ea_hbm.at[pl.ds(base, per_w)], ea_v)
        pltpu.sync_copy(ea_v, acc_sh.at[row_v], add=True)
        plsc.subcore_barrier()
        pltpu.sync_copy(acc_sh.at[pl.ds(s * _NPT, _NPT)], zb_v)
        pltpu.sync_copy(zb_v, out_hbm.at[pl.ds(c * _NP + s * _NPT, _NPT)])

    return k(row, edge_attr).reshape(_NC, _NP)


def _sc_edge(u, row, col, edge_attr):
    d = 24
    e_total = edge_attr.shape[0]
    per_w = e_total // _NW
    nchunks = per_w // _K
    nfull = d // 16
    ntail = d % 16

    @functools.partial(
        pl.kernel,
        out_type=jax.ShapeDtypeStruct((_NC, _NP, d), jnp.float32),
        mesh=_mesh(),
        scratch_types=[
            pltpu.VMEM((_K,), jnp.int32),
            pltpu.VMEM((_K,), jnp.int32),
            pltpu.VMEM((_K,), jnp.float32),
            pltpu.VMEM((_K, d), jnp.float32),
            pltpu.VMEM_SHARED((_NP, d), jnp.float32),
        ],
        compiler_params=_SC_PARAMS,
    )
    def k(u_hbm, row_hbm, col_hbm, ea_hbm, out_hbm, row_v, col_v, ea_v, rows_v, acc_sh):
        c = lax.axis_index("c")
        s = lax.axis_index("s")
        w = c * _NS + s
        zeros16 = jnp.zeros((16,), jnp.float32)

        def zrow(r, carry):
            for f in range(nfull):
                rows_v[r, pl.ds(f * 16, 16)] = zeros16
            if ntail:
                rows_v[r, pl.ds(d - 16, 16)] = zeros16
            return carry

        lax.fori_loop(0, _K, zrow, 0)
        nb = s * _NPT
        off = 0
        while off < _NPT:
            sz = min(_K, _NPT - off)
            pltpu.sync_copy(rows_v.at[pl.ds(0, sz)], acc_sh.at[pl.ds(nb + off, sz)])
            off += sz
        plsc.subcore_barrier()

        def chunk(i, carry):
            base = w * per_w + i * _K
            pltpu.sync_copy(row_hbm.at[pl.ds(base, _K)], row_v)
            pltpu.sync_copy(col_hbm.at[pl.ds(base, _K)], col_v)
            pltpu.sync_copy(ea_hbm.at[pl.ds(base, _K)], ea_v)
            pltpu.sync_copy(u_hbm.at[row_v], rows_v)

            def scale(e, carry2):
                er = jnp.full((16,), e, jnp.int32)
                sc = plsc.load_gather(ea_v, [er])
                offs = [f * 16 for f in range(nfull)]
                if ntail:
                    offs.append(d - 16)
                vals = [rows_v[e, pl.ds(o, 16)] * sc for o in offs]
                for o, v in zip(offs, vals):
                    rows_v[e, pl.ds(o, 16)] = v
                return carry2

            lax.fori_loop(0, _K, scale, 0)
            pltpu.sync_copy(rows_v, acc_sh.at[col_v], add=True)
            return carry

        lax.fori_loop(0, nchunks, chunk, 0)
        plsc.subcore_barrier()
        off = 0
        while off < _NPT:
            sz = min(_K, _NPT - off)
            pltpu.sync_copy(acc_sh.at[pl.ds(nb + off, sz)], rows_v.at[pl.ds(0, sz)])
            pltpu.sync_copy(rows_v.at[pl.ds(0, sz)],
                            out_hbm.at[c, pl.ds(nb + off, sz)])
            off += sz

    return k(u, row, col, edge_attr)


_BN = 1000


def _tc1(x, w10, w11, b1):
    n, d_in = x.shape
    d_h = w10.shape[1]

    def body(x_ref, w10_ref, w11_ref, b1_ref, h0_ref, xw1_ref):
        xb = x_ref[...]
        h0_ref[...] = jnp.dot(xb, w10_ref[...],
                              preferred_element_type=jnp.float32) + b1_ref[...]
        xw1_ref[...] = jnp.dot(xb, w11_ref[...],
                               preferred_element_type=jnp.float32)

    return pl.pallas_call(
        body,
        grid=(n // _BN,),
        in_specs=[
            pl.BlockSpec((_BN, d_in), lambda i: (i, 0)),
            pl.BlockSpec((d_in, d_h), lambda i: (0, 0)),
            pl.BlockSpec((d_in, d_h), lambda i: (0, 0)),
            pl.BlockSpec((1, d_h), lambda i: (0, 0)),
        ],
        out_specs=[
            pl.BlockSpec((_BN, d_h), lambda i: (i, 0)),
            pl.BlockSpec((_BN, d_h), lambda i: (i, 0)),
        ],
        out_shape=[
            jax.ShapeDtypeStruct((n, d_h), jnp.float32),
            jax.ShapeDtypeStruct((n, d_h), jnp.float32),
        ],
    )(x, w10, w11, b1)


def _tc2(degt, xw1):
    n, d_h = xw1.shape

    dhalf = d_h // 2
    dpad = dhalf + (-dhalf) % 8 + (8 if dhalf % 8 == 0 else 0)
    dpad = dhalf + 4

    def body(degt_ref, xw1_ref, dis_ref, u1a_ref, u1b_ref):
        deg = degt_ref[:, 0:1] + degt_ref[:, 1:2]
        dis2 = jnp.where(deg > 0, lax.rsqrt(deg), 0.0)
        dis_ref[...] = dis2
        u1 = dis2 * xw1_ref[...]
        zpad = jnp.zeros((u1.shape[0], dpad - dhalf), jnp.float32)
        u1a_ref[...] = jnp.concatenate([u1[:, :dhalf], zpad], axis=1)
        u1b_ref[...] = jnp.concatenate([u1[:, dhalf:], zpad], axis=1)

    return pl.pallas_call(
        body,
        grid=(n // _BN,),
        in_specs=[
            pl.BlockSpec((_BN, 2), lambda i: (i, 0)),
            pl.BlockSpec((_BN, d_h), lambda i: (i, 0)),
        ],
        out_specs=[
            pl.BlockSpec((_BN, 1), lambda i: (i, 0)),
            pl.BlockSpec((_BN, dpad), lambda i: (i, 0)),
            pl.BlockSpec((_BN, dpad), lambda i: (i, 0)),
        ],
        out_shape=[
            jax.ShapeDtypeStruct((n, 1), jnp.float32),
            jax.ShapeDtypeStruct((n, dpad), jnp.float32),
            jax.ShapeDtypeStruct((n, dpad), jnp.float32),
        ],
    )(degt, xw1)


def _tc3(h0, p1a, p1b, dis, w20, w21, b2):
    n, d_h = h0.shape
    d_out = w20.shape[1]
    dhalf = d_h // 2
    dpad = dhalf + 4
    opad = d_out + 4

    def body(h0_ref, p1a_ref, p1b_ref, dis_ref, w20_ref, w21_ref, b2_ref,
             h20_ref, u2_ref):
        s1 = jnp.concatenate([(p1a_ref[0] + p1a_ref[1])[:, :dhalf],
                              (p1b_ref[0] + p1b_ref[1])[:, :dhalf]], axis=1)
        dis2 = dis_ref[...]
        h = jnp.maximum(h0_ref[...] - dis2 * s1, 0.0)
        h20_ref[...] = jnp.dot(h, w20_ref[...],
                               preferred_element_type=jnp.float32) + b2_ref[...]
        u2 = dis2 * jnp.dot(h, w21_ref[...],
                            preferred_element_type=jnp.float32)
        zpad = jnp.zeros((u2.shape[0], opad - d_out), jnp.float32)
        u2_ref[...] = jnp.concatenate([u2, zpad], axis=1)

    return pl.pallas_call(
        body,
        grid=(n // _BN,),
        in_specs=[
            pl.BlockSpec((_BN, d_h), lambda i: (i, 0)),
            pl.BlockSpec((2, _BN, dpad), lambda i: (0, i, 0)),
            pl.BlockSpec((2, _BN, dpad), lambda i: (0, i, 0)),
            pl.BlockSpec((_BN, 1), lambda i: (i, 0)),
            pl.BlockSpec((d_h, d_out), lambda i: (0, 0)),
            pl.BlockSpec((d_h, d_out), lambda i: (0, 0)),
            pl.BlockSpec((1, d_out), lambda i: (0, 0)),
        ],
        out_specs=[
            pl.BlockSpec((_BN, d_out), lambda i: (i, 0)),
            pl.BlockSpec((_BN, opad), lambda i: (i, 0)),
        ],
        out_shape=[
            jax.ShapeDtypeStruct((n, d_out), jnp.float32),
            jax.ShapeDtypeStruct((n, opad), jnp.float32),
        ],
    )(h0, p1a, p1b, dis, w20, w21, b2)


def _tc4(h20, p2, dis, batch2d, wf, bf, nb):
    n, d_out = h20.shape
    n_cls = wf.shape[1]

    def body(h20_ref, p2_ref, dis_ref, b_ref, wf_ref, bf_ref, out_ref, g_acc):
        i = pl.program_id(0)

        @pl.when(i == 0)
        def _():
            g_acc[...] = jnp.zeros_like(g_acc)

        s2 = (p2_ref[0] + p2_ref[1])[:, :d_out]
        h2 = jnp.maximum(h20_ref[...] - dis_ref[...] * s2, 0.0)
        bids = lax.broadcasted_iota(jnp.int32, (_BN, nb), 1)
        oh = (b_ref[...] == bids).astype(jnp.float32)
        g_acc[...] += lax.dot_general(oh, h2, (((0,), (0,)), ((), ())),
                                      preferred_element_type=jnp.float32)

        @pl.when(i == pl.num_programs(0) - 1)
        def _():
            logits = jnp.dot(g_acc[...], wf_ref[...],
                             preferred_element_type=jnp.float32) + bf_ref[...]
            m = jnp.max(logits, axis=1, keepdims=True)
            lse = jnp.log(jnp.sum(jnp.exp(logits - m), axis=1, keepdims=True)) + m
            out_ref[...] = logits - lse

    return pl.pallas_call(
        body,
        grid=(n // _BN,),
        in_specs=[
            pl.BlockSpec((_BN, d_out), lambda i: (i, 0)),
            pl.BlockSpec((2, _BN, d_out + 4), lambda i: (0, i, 0)),
            pl.BlockSpec((_BN, 1), lambda i: (i, 0)),
            pl.BlockSpec((_BN, 1), lambda i: (i, 0)),
            pl.BlockSpec((d_out, n_cls), lambda i: (0, 0)),
            pl.BlockSpec((1, n_cls), lambda i: (0, 0)),
        ],
        out_specs=pl.BlockSpec((nb, n_cls), lambda i: (0, 0)),
        out_shape=jax.ShapeDtypeStruct((nb, n_cls), jnp.float32),
        scratch_shapes=[pltpu.VMEM((nb, d_out), jnp.float32)],
    )(h20, p2, dis, batch2d, wf, bf)


def kernel(x, edge_attr, W1_0, W1_1, b1, W2_0, W2_1, b2, Wf, bf,
           edge_index, batch, y):
    n = x.shape[0]
    nb = y.shape[0]
    row = edge_index[0]
    col = edge_index[1]

    h0, xw1 = _tc1(x, W1_0, W1_1, b1.reshape(1, -1))
    degp = _sc_deg(row, edge_attr)
    dis, u1a, u1b = _tc2(degp[:, :n].T, xw1)
    p1a = _sc_edge(u1a, row, col, edge_attr)
    p1b = _sc_edge(u1b, row, col, edge_attr)
    h20, u2 = _tc3(h0, p1a, p1b, dis, W2_0, W2_1, b2.reshape(1, -1))
    p2 = _sc_edge(u2, row, col, edge_attr)
    out = _tc4(h20, p2, dis, batch.reshape(-1, 1), Wf,
               bf.reshape(1, -1), nb)
    return out


# speedup vs baseline: 15.8868x; 1.0622x over previous
import functools

import jax
import jax.numpy as jnp
from jax import lax
from jax.experimental import pallas as pl
from jax.experimental.pallas import tpu as pltpu
from jax.experimental.pallas import tpu_sc as plsc

_NC, _NS = 2, 16
_NW = _NC * _NS
_NPT = 3136
_NP = _NPT * _NS
_K = 1000


def _mesh():
    return plsc.VectorSubcoreMesh(core_axis_name="c", subcore_axis_name="s")


_SC_PARAMS = pltpu.CompilerParams(needs_layout_passes=False,
                                  use_tc_tiling_on_sc=False)


def _sc_deg(row, edge_attr):
    e_total = edge_attr.shape[0]
    per_w = e_total // _NW

    @functools.partial(
        pl.kernel,
        out_type=jax.ShapeDtypeStruct((_NC * _NP,), jnp.float32),
        mesh=_mesh(),
        scratch_types=[
            pltpu.VMEM((per_w,), jnp.int32),
            pltpu.VMEM((per_w,), jnp.float32),
            pltpu.VMEM((_NPT,), jnp.float32),
            pltpu.VMEM_SHARED((_NP,), jnp.float32),
        ],
        compiler_params=_SC_PARAMS,
    )
    def k(row_hbm, ea_hbm, out_hbm, row_v, ea_v, zb_v, acc_sh):
        c = lax.axis_index("c")
        s = lax.axis_index("s")
        w = c * _NS + s

        def zb(i, carry):
            zb_v[pl.ds(i * 16, 16)] = jnp.zeros((16,), jnp.float32)
            return carry

        lax.fori_loop(0, _NPT // 16, zb, 0)
        pltpu.sync_copy(zb_v, acc_sh.at[pl.ds(s * _NPT, _NPT)])
        plsc.subcore_barrier()

        base = w * per_w
        pltpu.sync_copy(row_hbm.at[pl.ds(base, per_w)], row_v)
        pltpu.sync_copy(ea_hbm.at[pl.ds(base, per_w)], ea_v)
        pltpu.sync_copy(ea_v, acc_sh.at[row_v], add=True)
        plsc.subcore_barrier()
        pltpu.sync_copy(acc_sh.at[pl.ds(s * _NPT, _NPT)], zb_v)
        pltpu.sync_copy(zb_v, out_hbm.at[pl.ds(c * _NP + s * _NPT, _NPT)])

    return k(row, edge_attr).reshape(_NC, _NP)


def _sc_edge(u, row, col, edge_attr):
    d = 24
    e_total = edge_attr.shape[0]
    per_w = e_total // _NW
    nchunks = per_w // _K
    nfull = d // 16
    ntail = d % 16

    @functools.partial(
        pl.kernel,
        out_type=jax.ShapeDtypeStruct((_NC, _NP, d), jnp.float32),
        mesh=_mesh(),
        scratch_types=[
            pltpu.VMEM((_K,), jnp.int32),
            pltpu.VMEM((_K,), jnp.int32),
            pltpu.VMEM((_K,), jnp.float32),
            pltpu.VMEM((_K, d), jnp.float32),
            pltpu.VMEM_SHARED((_NP, d), jnp.float32),
        ],
        compiler_params=_SC_PARAMS,
    )
    def k(u_hbm, row_hbm, col_hbm, ea_hbm, out_hbm, row_v, col_v, ea_v, rows_v, acc_sh):
        c = lax.axis_index("c")
        s = lax.axis_index("s")
        w = c * _NS + s
        zeros16 = jnp.zeros((16,), jnp.float32)

        def zrow(r, carry):
            for f in range(nfull):
                rows_v[r, pl.ds(f * 16, 16)] = zeros16
            if ntail:
                rows_v[r, pl.ds(d - 16, 16)] = zeros16
            return carry

        lax.fori_loop(0, _K, zrow, 0)
        nb = s * _NPT
        off = 0
        while off < _NPT:
            sz = min(_K, _NPT - off)
            pltpu.sync_copy(rows_v.at[pl.ds(0, sz)], acc_sh.at[pl.ds(nb + off, sz)])
            off += sz
        plsc.subcore_barrier()

        def chunk(i, carry):
            base = w * per_w + i * _K
            pltpu.sync_copy(row_hbm.at[pl.ds(base, _K)], row_v)
            pltpu.sync_copy(col_hbm.at[pl.ds(base, _K)], col_v)
            pltpu.sync_copy(ea_hbm.at[pl.ds(base, _K)], ea_v)
            pltpu.sync_copy(u_hbm.at[row_v], rows_v)

            def scale(e, carry2):
                er = jnp.full((16,), e, jnp.int32)
                sc = plsc.load_gather(ea_v, [er])
                offs = [f * 16 for f in range(nfull)]
                if ntail:
                    offs.append(d - 16)
                vals = [rows_v[e, pl.ds(o, 16)] * sc for o in offs]
                for o, v in zip(offs, vals):
                    rows_v[e, pl.ds(o, 16)] = v
                return carry2

            lax.fori_loop(0, _K, scale, 0, unroll=8)
            pltpu.sync_copy(rows_v, acc_sh.at[col_v], add=True)
            return carry

        lax.fori_loop(0, nchunks, chunk, 0)
        plsc.subcore_barrier()
        off = 0
        while off < _NPT:
            sz = min(_K, _NPT - off)
            pltpu.sync_copy(acc_sh.at[pl.ds(nb + off, sz)], rows_v.at[pl.ds(0, sz)])
            pltpu.sync_copy(rows_v.at[pl.ds(0, sz)],
                            out_hbm.at[c, pl.ds(nb + off, sz)])
            off += sz

    return k(u, row, col, edge_attr)


_BN = 1000


def _tc1(x, w10, w11, b1):
    n, d_in = x.shape
    d_h = w10.shape[1]

    def body(x_ref, w10_ref, w11_ref, b1_ref, h0_ref, xw1_ref):
        xb = x_ref[...]
        h0_ref[...] = jnp.dot(xb, w10_ref[...],
                              preferred_element_type=jnp.float32) + b1_ref[...]
        xw1_ref[...] = jnp.dot(xb, w11_ref[...],
                               preferred_element_type=jnp.float32)

    return pl.pallas_call(
        body,
        grid=(n // _BN,),
        in_specs=[
            pl.BlockSpec((_BN, d_in), lambda i: (i, 0)),
            pl.BlockSpec((d_in, d_h), lambda i: (0, 0)),
            pl.BlockSpec((d_in, d_h), lambda i: (0, 0)),
            pl.BlockSpec((1, d_h), lambda i: (0, 0)),
        ],
        out_specs=[
            pl.BlockSpec((_BN, d_h), lambda i: (i, 0)),
            pl.BlockSpec((_BN, d_h), lambda i: (i, 0)),
        ],
        out_shape=[
            jax.ShapeDtypeStruct((n, d_h), jnp.float32),
            jax.ShapeDtypeStruct((n, d_h), jnp.float32),
        ],
    )(x, w10, w11, b1)


def _tc2(degt, xw1):
    n, d_h = xw1.shape

    dhalf = d_h // 2
    dpad = dhalf + (-dhalf) % 8 + (8 if dhalf % 8 == 0 else 0)
    dpad = dhalf + 4

    def body(degt_ref, xw1_ref, dis_ref, u1a_ref, u1b_ref):
        deg = degt_ref[:, 0:1] + degt_ref[:, 1:2]
        dis2 = jnp.where(deg > 0, lax.rsqrt(deg), 0.0)
        dis_ref[...] = dis2
        u1 = dis2 * xw1_ref[...]
        zpad = jnp.zeros((u1.shape[0], dpad - dhalf), jnp.float32)
        u1a_ref[...] = jnp.concatenate([u1[:, :dhalf], zpad], axis=1)
        u1b_ref[...] = jnp.concatenate([u1[:, dhalf:], zpad], axis=1)

    return pl.pallas_call(
        body,
        grid=(n // _BN,),
        in_specs=[
            pl.BlockSpec((_BN, 2), lambda i: (i, 0)),
            pl.BlockSpec((_BN, d_h), lambda i: (i, 0)),
        ],
        out_specs=[
            pl.BlockSpec((_BN, 1), lambda i: (i, 0)),
            pl.BlockSpec((_BN, dpad), lambda i: (i, 0)),
            pl.BlockSpec((_BN, dpad), lambda i: (i, 0)),
        ],
        out_shape=[
            jax.ShapeDtypeStruct((n, 1), jnp.float32),
            jax.ShapeDtypeStruct((n, dpad), jnp.float32),
            jax.ShapeDtypeStruct((n, dpad), jnp.float32),
        ],
    )(degt, xw1)


def _tc3(h0, p1a, p1b, dis, w20, w21, b2):
    n, d_h = h0.shape
    d_out = w20.shape[1]
    dhalf = d_h // 2
    dpad = dhalf + 4
    opad = d_out + 4

    def body(h0_ref, p1a_ref, p1b_ref, dis_ref, w20_ref, w21_ref, b2_ref,
             h20_ref, u2_ref):
        s1 = jnp.concatenate([(p1a_ref[0] + p1a_ref[1])[:, :dhalf],
                              (p1b_ref[0] + p1b_ref[1])[:, :dhalf]], axis=1)
        dis2 = dis_ref[...]
        h = jnp.maximum(h0_ref[...] - dis2 * s1, 0.0)
        h20_ref[...] = jnp.dot(h, w20_ref[...],
                               preferred_element_type=jnp.float32) + b2_ref[...]
        u2 = dis2 * jnp.dot(h, w21_ref[...],
                            preferred_element_type=jnp.float32)
        zpad = jnp.zeros((u2.shape[0], opad - d_out), jnp.float32)
        u2_ref[...] = jnp.concatenate([u2, zpad], axis=1)

    return pl.pallas_call(
        body,
        grid=(n // _BN,),
        in_specs=[
            pl.BlockSpec((_BN, d_h), lambda i: (i, 0)),
            pl.BlockSpec((2, _BN, dpad), lambda i: (0, i, 0)),
            pl.BlockSpec((2, _BN, dpad), lambda i: (0, i, 0)),
            pl.BlockSpec((_BN, 1), lambda i: (i, 0)),
            pl.BlockSpec((d_h, d_out), lambda i: (0, 0)),
            pl.BlockSpec((d_h, d_out), lambda i: (0, 0)),
            pl.BlockSpec((1, d_out), lambda i: (0, 0)),
        ],
        out_specs=[
            pl.BlockSpec((_BN, d_out), lambda i: (i, 0)),
            pl.BlockSpec((_BN, opad), lambda i: (i, 0)),
        ],
        out_shape=[
            jax.ShapeDtypeStruct((n, d_out), jnp.float32),
            jax.ShapeDtypeStruct((n, opad), jnp.float32),
        ],
    )(h0, p1a, p1b, dis, w20, w21, b2)


def _tc4(h20, p2, dis, batch2d, wf, bf, nb):
    n, d_out = h20.shape
    n_cls = wf.shape[1]

    def body(h20_ref, p2_ref, dis_ref, b_ref, wf_ref, bf_ref, out_ref, g_acc):
        i = pl.program_id(0)

        @pl.when(i == 0)
        def _():
            g_acc[...] = jnp.zeros_like(g_acc)

        s2 = (p2_ref[0] + p2_ref[1])[:, :d_out]
        h2 = jnp.maximum(h20_ref[...] - dis_ref[...] * s2, 0.0)
        bids = lax.broadcasted_iota(jnp.int32, (_BN, nb), 1)
        oh = (b_ref[...] == bids).astype(jnp.float32)
        g_acc[...] += lax.dot_general(oh, h2, (((0,), (0,)), ((), ())),
                                      preferred_element_type=jnp.float32)

        @pl.when(i == pl.num_programs(0) - 1)
        def _():
            logits = jnp.dot(g_acc[...], wf_ref[...],
                             preferred_element_type=jnp.float32) + bf_ref[...]
            m = jnp.max(logits, axis=1, keepdims=True)
            lse = jnp.log(jnp.sum(jnp.exp(logits - m), axis=1, keepdims=True)) + m
            out_ref[...] = logits - lse

    return pl.pallas_call(
        body,
        grid=(n // _BN,),
        in_specs=[
            pl.BlockSpec((_BN, d_out), lambda i: (i, 0)),
            pl.BlockSpec((2, _BN, d_out + 4), lambda i: (0, i, 0)),
            pl.BlockSpec((_BN, 1), lambda i: (i, 0)),
            pl.BlockSpec((_BN, 1), lambda i: (i, 0)),
            pl.BlockSpec((d_out, n_cls), lambda i: (0, 0)),
            pl.BlockSpec((1, n_cls), lambda i: (0, 0)),
        ],
        out_specs=pl.BlockSpec((nb, n_cls), lambda i: (0, 0)),
        out_shape=jax.ShapeDtypeStruct((nb, n_cls), jnp.float32),
        scratch_shapes=[pltpu.VMEM((nb, d_out), jnp.float32)],
    )(h20, p2, dis, batch2d, wf, bf)


def kernel(x, edge_attr, W1_0, W1_1, b1, W2_0, W2_1, b2, Wf, bf,
           edge_index, batch, y):
    n = x.shape[0]
    nb = y.shape[0]
    row = edge_index[0]
    col = edge_index[1]

    h0, xw1 = _tc1(x, W1_0, W1_1, b1.reshape(1, -1))
    degp = _sc_deg(row, edge_attr)
    dis, u1a, u1b = _tc2(degp[:, :n].T, xw1)
    p1a = _sc_edge(u1a, row, col, edge_attr)
    p1b = _sc_edge(u1b, row, col, edge_attr)
    h20, u2 = _tc3(h0, p1a, p1b, dis, W2_0, W2_1, b2.reshape(1, -1))
    p2 = _sc_edge(u2, row, col, edge_attr)
    out = _tc4(h20, p2, dis, batch.reshape(-1, 1), Wf,
               bf.reshape(1, -1), nb)
    return out


# speedup vs baseline: 18.6749x; 1.1755x over previous
import functools

import jax
import jax.numpy as jnp
from jax import lax
from jax.experimental import pallas as pl
from jax.experimental.pallas import tpu as pltpu
from jax.experimental.pallas import tpu_sc as plsc

_NC, _NS = 2, 16
_NW = _NC * _NS
_NPT = 3136
_NP = _NPT * _NS
_K = 1000
_D = 24


def _mesh():
    return plsc.VectorSubcoreMesh(core_axis_name="c", subcore_axis_name="s")


_SC_PARAMS = pltpu.CompilerParams(needs_layout_passes=False,
                                  use_tc_tiling_on_sc=False)


def _sc_deg(row, edge_attr):
    e_total = edge_attr.shape[0]
    per_w = e_total // _NW

    @functools.partial(
        pl.kernel,
        out_type=jax.ShapeDtypeStruct((_NC * _NP,), jnp.float32),
        mesh=_mesh(),
        scratch_types=[
            pltpu.VMEM((per_w,), jnp.int32),
            pltpu.VMEM((per_w,), jnp.float32),
            pltpu.VMEM((_NPT,), jnp.float32),
            pltpu.VMEM_SHARED((_NP,), jnp.float32),
        ],
        compiler_params=_SC_PARAMS,
    )
    def k(row_hbm, ea_hbm, out_hbm, row_v, ea_v, zb_v, acc_sh):
        c = lax.axis_index("c")
        s = lax.axis_index("s")
        w = c * _NS + s

        def zb(i, carry):
            zb_v[pl.ds(i * 16, 16)] = jnp.zeros((16,), jnp.float32)
            return carry

        lax.fori_loop(0, _NPT // 16, zb, 0)
        pltpu.sync_copy(zb_v, acc_sh.at[pl.ds(s * _NPT, _NPT)])
        plsc.subcore_barrier()

        base = w * per_w
        pltpu.sync_copy(row_hbm.at[pl.ds(base, per_w)], row_v)
        pltpu.sync_copy(ea_hbm.at[pl.ds(base, per_w)], ea_v)
        pltpu.sync_copy(ea_v, acc_sh.at[row_v], add=True)
        plsc.subcore_barrier()
        pltpu.sync_copy(acc_sh.at[pl.ds(s * _NPT, _NPT)], zb_v)
        pltpu.sync_copy(zb_v, out_hbm.at[pl.ds(c * _NP + s * _NPT, _NPT)])

    return k(row, edge_attr).reshape(_NC, _NP)


def _sc_edge_multi(us, row, col, edge_attr):
    nph = len(us)
    d = _D
    e_total = edge_attr.shape[0]
    per_w = e_total // _NW
    nchunks = per_w // _K
    npairs = (nchunks - 1) // 2
    nfull = d // 16
    ntail = d % 16
    assert nchunks == 2 * npairs + 1

    @functools.partial(
        pl.kernel,
        out_type=jax.ShapeDtypeStruct((_NC, nph, _NP, d), jnp.float32),
        mesh=_mesh(),
        scratch_types=[
            pltpu.VMEM((_K,), jnp.int32),
            pltpu.VMEM((_K,), jnp.int32),
            pltpu.VMEM((_K,), jnp.int32),
            pltpu.VMEM((_K,), jnp.int32),
            pltpu.VMEM((_K,), jnp.float32),
            pltpu.VMEM((_K,), jnp.float32),
            pltpu.VMEM((_K, _D), jnp.float32),
            pltpu.VMEM((_K, _D), jnp.float32),
            pltpu.VMEM_SHARED((_NP, _D), jnp.float32),
            pltpu.SemaphoreType.DMA,
            pltpu.SemaphoreType.DMA,
            pltpu.SemaphoreType.DMA,
            pltpu.SemaphoreType.DMA,
        ],
        compiler_params=_SC_PARAMS,
    )
    def k(*refs):
        us_hbm = refs[:nph]
        row_hbm, col_hbm, ea_hbm, out_hbm = refs[nph:nph + 4]
        (row_vA, row_vB, col_vA, col_vB, ea_vA, ea_vB, rows_vA, rows_vB,
         acc_sh, gsemA, gsemB, ssemA, ssemB) = refs[nph + 4:]
        c = lax.axis_index("c")
        s = lax.axis_index("s")
        w = c * _NS + s
        nb = s * _NPT
        zeros16 = jnp.zeros((16,), jnp.float32)

        def zero_rows(rv):
            def zr(r, carry):
                for f in range(nfull):
                    rv[r, pl.ds(f * 16, 16)] = zeros16
                if ntail:
                    rv[r, pl.ds(d - 16, 16)] = zeros16
                return carry
            lax.fori_loop(0, _K, zr, 0, unroll=8)

        def scale_rows(rv, eav):
            def sb(e, carry):
                er = jnp.full((16,), e, jnp.int32)
                scv = plsc.load_gather(eav, [er])
                offs = [f * 16 for f in range(nfull)]
                if ntail:
                    offs.append(d - 16)
                vals = [rv[e, pl.ds(o, 16)] * scv for o in offs]
                for o, v in zip(offs, vals):
                    rv[e, pl.ds(o, 16)] = v
                return carry
            lax.fori_loop(0, _K, sb, 0, unroll=8)

        for p in range(nph):
            u_hbm = us_hbm[p]

            def idx_load(i, row_v, col_v, ea_v):
                base = w * per_w + i * _K
                pltpu.sync_copy(row_hbm.at[pl.ds(base, _K)], row_v)
                pltpu.sync_copy(col_hbm.at[pl.ds(base, _K)], col_v)
                pltpu.sync_copy(ea_hbm.at[pl.ds(base, _K)], ea_v)

            def gather_start(row_v, rows_v, gsem):
                pltpu.async_copy(u_hbm.at[row_v], rows_v, gsem)

            def gather_wait(row_v, rows_v, gsem):
                pltpu.make_async_copy(u_hbm.at[row_v], rows_v, gsem).wait()

            def scatter_start(rows_v, col_v, ssem):
                pltpu.async_copy(rows_v, acc_sh.at[col_v], ssem, add=True)

            def scatter_wait(rows_v, col_v, ssem):
                pltpu.make_async_copy(rows_v, acc_sh.at[col_v], ssem).wait()

            zero_rows(rows_vA)
            off = 0
            while off < _NPT:
                sz = min(_K, _NPT - off)
                pltpu.sync_copy(rows_vA.at[pl.ds(0, sz)],
                                acc_sh.at[pl.ds(nb + off, sz)])
                off += sz
            plsc.subcore_barrier()

            idx_load(0, row_vA, col_vA, ea_vA)
            gather_start(row_vA, rows_vA, gsemA)

            def pair(i2, carry):
                i_b = 2 * i2 + 1

                @pl.when(i2 > 0)
                def _():
                    scatter_wait(rows_vB, col_vB, ssemB)

                idx_load(i_b, row_vB, col_vB, ea_vB)
                gather_start(row_vB, rows_vB, gsemB)
                gather_wait(row_vA, rows_vA, gsemA)
                scale_rows(rows_vA, ea_vA)
                scatter_start(rows_vA, col_vA, ssemA)
                gather_wait(row_vB, rows_vB, gsemB)
                scale_rows(rows_vB, ea_vB)
                scatter_start(rows_vB, col_vB, ssemB)
                scatter_wait(rows_vA, col_vA, ssemA)
                idx_load(i_b + 1, row_vA, col_vA, ea_vA)
                gather_start(row_vA, rows_vA, gsemA)
                return carry

            lax.fori_loop(0, npairs, pair, 0)
            gather_wait(row_vA, rows_vA, gsemA)
            scale_rows(rows_vA, ea_vA)
            scatter_start(rows_vA, col_vA, ssemA)
            scatter_wait(rows_vA, col_vA, ssemA)
            scatter_wait(rows_vB, col_vB, ssemB)
            plsc.subcore_barrier()

            off = 0
            while off < _NPT:
                sz = min(_K, _NPT - off)
                pltpu.sync_copy(acc_sh.at[pl.ds(nb + off, sz)],
                                rows_vA.at[pl.ds(0, sz)])
                pltpu.sync_copy(rows_vA.at[pl.ds(0, sz)],
                                out_hbm.at[c, p, pl.ds(nb + off, sz)])
                off += sz
            plsc.subcore_barrier()

    return k(*us, row, col, edge_attr)


_BN = 1000


def _tc12(x, w10, w11, b1, degt):
    n, d_in = x.shape
    d_h = w10.shape[1]
    dhalf = d_h // 2
    dpad = _D

    def body(x_ref, w10_ref, w11_ref, b1_ref, degt_ref,
             h0_ref, dis_ref, u1a_ref, u1b_ref):
        xb = x_ref[...]
        h0_ref[...] = jnp.dot(xb, w10_ref[...],
                              preferred_element_type=jnp.float32) + b1_ref[...]
        xw1 = jnp.dot(xb, w11_ref[...], preferred_element_type=jnp.float32)
        deg = degt_ref[:, 0:1] + degt_ref[:, 1:2]
        dis2 = jnp.where(deg > 0, lax.rsqrt(deg), 0.0)
        dis_ref[...] = dis2
        u1 = dis2 * xw1
        zpad = jnp.zeros((u1.shape[0], dpad - dhalf), jnp.float32)
        u1a_ref[...] = jnp.concatenate([u1[:, :dhalf], zpad], axis=1)
        u1b_ref[...] = jnp.concatenate([u1[:, dhalf:], zpad], axis=1)

    return pl.pallas_call(
        body,
        grid=(n // _BN,),
        in_specs=[
            pl.BlockSpec((_BN, d_in), lambda i: (i, 0)),
            pl.BlockSpec((d_in, d_h), lambda i: (0, 0)),
            pl.BlockSpec((d_in, d_h), lambda i: (0, 0)),
            pl.BlockSpec((1, d_h), lambda i: (0, 0)),
            pl.BlockSpec((_BN, 2), lambda i: (i, 0)),
        ],
        out_specs=[
            pl.BlockSpec((_BN, d_h), lambda i: (i, 0)),
            pl.BlockSpec((_BN, 1), lambda i: (i, 0)),
            pl.BlockSpec((_BN, dpad), lambda i: (i, 0)),
            pl.BlockSpec((_BN, dpad), lambda i: (i, 0)),
        ],
        out_shape=[
            jax.ShapeDtypeStruct((n, d_h), jnp.float32),
            jax.ShapeDtypeStruct((n, 1), jnp.float32),
            jax.ShapeDtypeStruct((n, dpad), jnp.float32),
            jax.ShapeDtypeStruct((n, dpad), jnp.float32),
        ],
    )(x, w10, w11, b1, degt)


def _tc3(h0, p1, dis, w20, w21, b2):
    n, d_h = h0.shape
    d_out = w20.shape[1]
    dhalf = d_h // 2
    opad = _D

    def body(h0_ref, p1_ref, dis_ref, w20_ref, w21_ref, b2_ref,
             h20_ref, u2_ref):
        s1 = jnp.concatenate([(p1_ref[0, 0] + p1_ref[1, 0])[:, :dhalf],
                              (p1_ref[0, 1] + p1_ref[1, 1])[:, :dhalf]],
                             axis=1)
        dis2 = dis_ref[...]
        h = jnp.maximum(h0_ref[...] - dis2 * s1, 0.0)
        h20_ref[...] = jnp.dot(h, w20_ref[...],
                               preferred_element_type=jnp.float32) + b2_ref[...]
        u2 = dis2 * jnp.dot(h, w21_ref[...],
                            preferred_element_type=jnp.float32)
        zpad = jnp.zeros((u2.shape[0], opad - d_out), jnp.float32)
        u2_ref[...] = jnp.concatenate([u2, zpad], axis=1)

    return pl.pallas_call(
        body,
        grid=(n // _BN,),
        in_specs=[
            pl.BlockSpec((_BN, d_h), lambda i: (i, 0)),
            pl.BlockSpec((2, 2, _BN, _D), lambda i: (0, 0, i, 0)),
            pl.BlockSpec((_BN, 1), lambda i: (i, 0)),
            pl.BlockSpec((d_h, d_out), lambda i: (0, 0)),
            pl.BlockSpec((d_h, d_out), lambda i: (0, 0)),
            pl.BlockSpec((1, d_out), lambda i: (0, 0)),
        ],
        out_specs=[
            pl.BlockSpec((_BN, d_out), lambda i: (i, 0)),
            pl.BlockSpec((_BN, opad), lambda i: (i, 0)),
        ],
        out_shape=[
            jax.ShapeDtypeStruct((n, d_out), jnp.float32),
            jax.ShapeDtypeStruct((n, opad), jnp.float32),
        ],
    )(h0, p1, dis, w20, w21, b2)


def _tc4(h20, p2, dis, batch2d, wf, bf, nb):
    n, d_out = h20.shape
    n_cls = wf.shape[1]

    def body(h20_ref, p2_ref, dis_ref, b_ref, wf_ref, bf_ref, out_ref, g_acc):
        i = pl.program_id(0)

        @pl.when(i == 0)
        def _():
            g_acc[...] = jnp.zeros_like(g_acc)

        s2 = (p2_ref[0, 0] + p2_ref[1, 0])[:, :d_out]
        h2 = jnp.maximum(h20_ref[...] - dis_ref[...] * s2, 0.0)
        bids = lax.broadcasted_iota(jnp.int32, (_BN, nb), 1)
        oh = (b_ref[...] == bids).astype(jnp.float32)
        g_acc[...] += lax.dot_general(oh, h2, (((0,), (0,)), ((), ())),
                                      preferred_element_type=jnp.float32)

        @pl.when(i == pl.num_programs(0) - 1)
        def _():
            logits = jnp.dot(g_acc[...], wf_ref[...],
                             preferred_element_type=jnp.float32) + bf_ref[...]
            m = jnp.max(logits, axis=1, keepdims=True)
            lse = jnp.log(jnp.sum(jnp.exp(logits - m), axis=1, keepdims=True)) + m
            out_ref[...] = logits - lse

    return pl.pallas_call(
        body,
        grid=(n // _BN,),
        in_specs=[
            pl.BlockSpec((_BN, d_out), lambda i: (i, 0)),
            pl.BlockSpec((2, 1, _BN, _D), lambda i: (0, 0, i, 0)),
            pl.BlockSpec((_BN, 1), lambda i: (i, 0)),
            pl.BlockSpec((_BN, 1), lambda i: (i, 0)),
            pl.BlockSpec((d_out, n_cls), lambda i: (0, 0)),
            pl.BlockSpec((1, n_cls), lambda i: (0, 0)),
        ],
        out_specs=pl.BlockSpec((nb, n_cls), lambda i: (0, 0)),
        out_shape=jax.ShapeDtypeStruct((nb, n_cls), jnp.float32),
        scratch_shapes=[pltpu.VMEM((nb, d_out), jnp.float32)],
    )(h20, p2, dis, batch2d, wf, bf)


def kernel(x, edge_attr, W1_0, W1_1, b1, W2_0, W2_1, b2, Wf, bf,
           edge_index, batch, y):
    n = x.shape[0]
    nb = y.shape[0]
    row = edge_index[0]
    col = edge_index[1]

    degp = _sc_deg(row, edge_attr)
    h0, dis, u1a, u1b = _tc12(x, W1_0, W1_1, b1.reshape(1, -1),
                              degp[:, :n].T)
    p1 = _sc_edge_multi((u1a, u1b), row, col, edge_attr)
    h20, u2 = _tc3(h0, p1, dis, W2_0, W2_1, b2.reshape(1, -1))
    p2 = _sc_edge_multi((u2,), row, col, edge_attr)
    out = _tc4(h20, p2, dis, batch.reshape(-1, 1), Wf,
               bf.reshape(1, -1), nb)
    return out


# speedup vs baseline: 18.6996x; 1.0013x over previous
import functools

import jax
import jax.numpy as jnp
from jax import lax
from jax.experimental import pallas as pl
from jax.experimental.pallas import tpu as pltpu
from jax.experimental.pallas import tpu_sc as plsc

_NC, _NS = 2, 16
_NW = _NC * _NS
_NPT = 3136
_NP = _NPT * _NS
_K = 1000
_D = 24


def _mesh():
    return plsc.VectorSubcoreMesh(core_axis_name="c", subcore_axis_name="s")


_SC_PARAMS = pltpu.CompilerParams(needs_layout_passes=False,
                                  use_tc_tiling_on_sc=False)


def _sc_deg(row, edge_attr):
    e_total = edge_attr.shape[0]
    per_w = e_total // _NW

    @functools.partial(
        pl.kernel,
        out_type=jax.ShapeDtypeStruct((_NC * _NP,), jnp.float32),
        mesh=_mesh(),
        scratch_types=[
            pltpu.VMEM((per_w,), jnp.int32),
            pltpu.VMEM((per_w,), jnp.float32),
            pltpu.VMEM((_NPT,), jnp.float32),
            pltpu.VMEM_SHARED((_NP,), jnp.float32),
        ],
        compiler_params=_SC_PARAMS,
    )
    def k(row_hbm, ea_hbm, out_hbm, row_v, ea_v, zb_v, acc_sh):
        c = lax.axis_index("c")
        s = lax.axis_index("s")
        w = c * _NS + s

        def zb(i, carry):
            zb_v[pl.ds(i * 16, 16)] = jnp.zeros((16,), jnp.float32)
            return carry

        lax.fori_loop(0, _NPT // 16, zb, 0)
        pltpu.sync_copy(zb_v, acc_sh.at[pl.ds(s * _NPT, _NPT)])
        plsc.subcore_barrier()

        base = w * per_w
        pltpu.sync_copy(row_hbm.at[pl.ds(base, per_w)], row_v)
        pltpu.sync_copy(ea_hbm.at[pl.ds(base, per_w)], ea_v)
        pltpu.sync_copy(ea_v, acc_sh.at[row_v], add=True)
        plsc.subcore_barrier()
        pltpu.sync_copy(acc_sh.at[pl.ds(s * _NPT, _NPT)], zb_v)
        pltpu.sync_copy(zb_v, out_hbm.at[pl.ds(c * _NP + s * _NPT, _NPT)])

    return k(row, edge_attr).reshape(_NC, _NP)


def _sc_edge_multi(us, row, col, edge_attr):
    nph = len(us)
    d = _D
    e_total = edge_attr.shape[0]
    per_w = e_total // _NW
    nchunks = per_w // _K
    npairs = (nchunks - 1) // 2
    nfull = d // 16
    ntail = d % 16
    assert nchunks == 2 * npairs + 1

    @functools.partial(
        pl.kernel,
        out_type=jax.ShapeDtypeStruct((_NC, nph, _NP, d), jnp.float32),
        mesh=_mesh(),
        scratch_types=[
            pltpu.VMEM((_K,), jnp.int32),
            pltpu.VMEM((_K,), jnp.int32),
            pltpu.VMEM((_K,), jnp.int32),
            pltpu.VMEM((_K,), jnp.int32),
            pltpu.VMEM((_K,), jnp.float32),
            pltpu.VMEM((_K,), jnp.float32),
            pltpu.VMEM((_K, _D), jnp.float32),
            pltpu.VMEM((_K, _D), jnp.float32),
            pltpu.VMEM_SHARED((_NP, _D), jnp.float32),
            pltpu.SemaphoreType.DMA,
            pltpu.SemaphoreType.DMA,
            pltpu.SemaphoreType.DMA,
            pltpu.SemaphoreType.DMA,
        ],
        compiler_params=_SC_PARAMS,
    )
    def k(*refs):
        us_hbm = refs[:nph]
        row_hbm, col_hbm, ea_hbm, out_hbm = refs[nph:nph + 4]
        (row_vA, row_vB, col_vA, col_vB, ea_vA, ea_vB, rows_vA, rows_vB,
         acc_sh, gsemA, gsemB, ssemA, ssemB) = refs[nph + 4:]
        c = lax.axis_index("c")
        s = lax.axis_index("s")
        w = c * _NS + s
        nb = s * _NPT
        zeros16 = jnp.zeros((16,), jnp.float32)

        def zero_rows(rv):
            def zr(r, carry):
                for f in range(nfull):
                    rv[r, pl.ds(f * 16, 16)] = zeros16
                if ntail:
                    rv[r, pl.ds(d - 16, 16)] = zeros16
                return carry
            lax.fori_loop(0, _K, zr, 0, unroll=8)

        def scale_rows(rv, eav):
            def sb(e, carry):
                er = jnp.full((16,), e, jnp.int32)
                scv = plsc.load_gather(eav, [er])
                offs = [f * 16 for f in range(nfull)]
                if ntail:
                    offs.append(d - 16)
                vals = [rv[e, pl.ds(o, 16)] * scv for o in offs]
                for o, v in zip(offs, vals):
                    rv[e, pl.ds(o, 16)] = v
                return carry
            lax.fori_loop(0, _K, sb, 0, unroll=10)

        for p in range(nph):
            u_hbm = us_hbm[p]

            def idx_load(i, row_v, col_v, ea_v):
                base = w * per_w + i * _K
                pltpu.sync_copy(row_hbm.at[pl.ds(base, _K)], row_v)
                pltpu.sync_copy(col_hbm.at[pl.ds(base, _K)], col_v)
                pltpu.sync_copy(ea_hbm.at[pl.ds(base, _K)], ea_v)

            def gather_start(row_v, rows_v, gsem):
                pltpu.async_copy(u_hbm.at[row_v], rows_v, gsem)

            def gather_wait(row_v, rows_v, gsem):
                pltpu.make_async_copy(u_hbm.at[row_v], rows_v, gsem).wait()

            def scatter_start(rows_v, col_v, ssem):
                pltpu.async_copy(rows_v, acc_sh.at[col_v], ssem, add=True)

            def scatter_wait(rows_v, col_v, ssem):
                pltpu.make_async_copy(rows_v, acc_sh.at[col_v], ssem).wait()

            zero_rows(rows_vA)
            off = 0
            while off < _NPT:
                sz = min(_K, _NPT - off)
                pltpu.sync_copy(rows_vA.at[pl.ds(0, sz)],
                                acc_sh.at[pl.ds(nb + off, sz)])
                off += sz
            plsc.subcore_barrier()

            idx_load(0, row_vA, col_vA, ea_vA)
            gather_start(row_vA, rows_vA, gsemA)

            def pair(i2, carry):
                i_b = 2 * i2 + 1

                @pl.when(i2 > 0)
                def _():
                    scatter_wait(rows_vB, col_vB, ssemB)

                idx_load(i_b, row_vB, col_vB, ea_vB)
                gather_start(row_vB, rows_vB, gsemB)
                gather_wait(row_vA, rows_vA, gsemA)
                scale_rows(rows_vA, ea_vA)
                scatter_start(rows_vA, col_vA, ssemA)
                gather_wait(row_vB, rows_vB, gsemB)
                scale_rows(rows_vB, ea_vB)
                scatter_start(rows_vB, col_vB, ssemB)
                scatter_wait(rows_vA, col_vA, ssemA)
                idx_load(i_b + 1, row_vA, col_vA, ea_vA)
                gather_start(row_vA, rows_vA, gsemA)
                return carry

            lax.fori_loop(0, npairs, pair, 0)
            gather_wait(row_vA, rows_vA, gsemA)
            scale_rows(rows_vA, ea_vA)
            scatter_start(rows_vA, col_vA, ssemA)
            scatter_wait(rows_vA, col_vA, ssemA)
            scatter_wait(rows_vB, col_vB, ssemB)
            plsc.subcore_barrier()

            off = 0
            while off < _NPT:
                sz = min(_K, _NPT - off)
                pltpu.sync_copy(acc_sh.at[pl.ds(nb + off, sz)],
                                rows_vA.at[pl.ds(0, sz)])
                pltpu.sync_copy(rows_vA.at[pl.ds(0, sz)],
                                out_hbm.at[c, p, pl.ds(nb + off, sz)])
                off += sz
            plsc.subcore_barrier()

    return k(*us, row, col, edge_attr)


_BN = 1000


def _tc12(x, w10, w11, b1, degt):
    n, d_in = x.shape
    d_h = w10.shape[1]
    dhalf = d_h // 2
    dpad = _D

    def body(x_ref, w10_ref, w11_ref, b1_ref, degt_ref,
             h0_ref, dis_ref, u1a_ref, u1b_ref):
        xb = x_ref[...]
        h0_ref[...] = jnp.dot(xb, w10_ref[...],
                              preferred_element_type=jnp.float32) + b1_ref[...]
        xw1 = jnp.dot(xb, w11_ref[...], preferred_element_type=jnp.float32)
        deg = degt_ref[:, 0:1] + degt_ref[:, 1:2]
        dis2 = jnp.where(deg > 0, lax.rsqrt(deg), 0.0)
        dis_ref[...] = dis2
        u1 = dis2 * xw1
        zpad = jnp.zeros((u1.shape[0], dpad - dhalf), jnp.float32)
        u1a_ref[...] = jnp.concatenate([u1[:, :dhalf], zpad], axis=1)
        u1b_ref[...] = jnp.concatenate([u1[:, dhalf:], zpad], axis=1)

    return pl.pallas_call(
        body,
        grid=(n // _BN,),
        in_specs=[
            pl.BlockSpec((_BN, d_in), lambda i: (i, 0)),
            pl.BlockSpec((d_in, d_h), lambda i: (0, 0)),
            pl.BlockSpec((d_in, d_h), lambda i: (0, 0)),
            pl.BlockSpec((1, d_h), lambda i: (0, 0)),
            pl.BlockSpec((_BN, 2), lambda i: (i, 0)),
        ],
        out_specs=[
            pl.BlockSpec((_BN, d_h), lambda i: (i, 0)),
            pl.BlockSpec((_BN, 1), lambda i: (i, 0)),
            pl.BlockSpec((_BN, dpad), lambda i: (i, 0)),
            pl.BlockSpec((_BN, dpad), lambda i: (i, 0)),
        ],
        out_shape=[
            jax.ShapeDtypeStruct((n, d_h), jnp.float32),
            jax.ShapeDtypeStruct((n, 1), jnp.float32),
            jax.ShapeDtypeStruct((n, dpad), jnp.float32),
            jax.ShapeDtypeStruct((n, dpad), jnp.float32),
        ],
    )(x, w10, w11, b1, degt)


def _tc3(h0, p1, dis, w20, w21, b2):
    n, d_h = h0.shape
    d_out = w20.shape[1]
    dhalf = d_h // 2
    opad = _D

    def body(h0_ref, p1_ref, dis_ref, w20_ref, w21_ref, b2_ref,
             h20_ref, u2_ref):
        s1 = jnp.concatenate([(p1_ref[0, 0] + p1_ref[1, 0])[:, :dhalf],
                              (p1_ref[0, 1] + p1_ref[1, 1])[:, :dhalf]],
                             axis=1)
        dis2 = dis_ref[...]
        h = jnp.maximum(h0_ref[...] - dis2 * s1, 0.0)
        h20_ref[...] = jnp.dot(h, w20_ref[...],
                               preferred_element_type=jnp.float32) + b2_ref[...]
        u2 = dis2 * jnp.dot(h, w21_ref[...],
                            preferred_element_type=jnp.float32)
        zpad = jnp.zeros((u2.shape[0], opad - d_out), jnp.float32)
        u2_ref[...] = jnp.concatenate([u2, zpad], axis=1)

    return pl.pallas_call(
        body,
        grid=(n // _BN,),
        in_specs=[
            pl.BlockSpec((_BN, d_h), lambda i: (i, 0)),
            pl.BlockSpec((2, 2, _BN, _D), lambda i: (0, 0, i, 0)),
            pl.BlockSpec((_BN, 1), lambda i: (i, 0)),
            pl.BlockSpec((d_h, d_out), lambda i: (0, 0)),
            pl.BlockSpec((d_h, d_out), lambda i: (0, 0)),
            pl.BlockSpec((1, d_out), lambda i: (0, 0)),
        ],
        out_specs=[
            pl.BlockSpec((_BN, d_out), lambda i: (i, 0)),
            pl.BlockSpec((_BN, opad), lambda i: (i, 0)),
        ],
        out_shape=[
            jax.ShapeDtypeStruct((n, d_out), jnp.float32),
            jax.ShapeDtypeStruct((n, opad), jnp.float32),
        ],
    )(h0, p1, dis, w20, w21, b2)


def _tc4(h20, p2, dis, batch2d, wf, bf, nb):
    n, d_out = h20.shape
    n_cls = wf.shape[1]

    def body(h20_ref, p2_ref, dis_ref, b_ref, wf_ref, bf_ref, out_ref, g_acc):
        i = pl.program_id(0)

        @pl.when(i == 0)
        def _():
            g_acc[...] = jnp.zeros_like(g_acc)

        s2 = (p2_ref[0, 0] + p2_ref[1, 0])[:, :d_out]
        h2 = jnp.maximum(h20_ref[...] - dis_ref[...] * s2, 0.0)
        bids = lax.broadcasted_iota(jnp.int32, (_BN, nb), 1)
        oh = (b_ref[...] == bids).astype(jnp.float32)
        g_acc[...] += lax.dot_general(oh, h2, (((0,), (0,)), ((), ())),
                                      preferred_element_type=jnp.float32)

        @pl.when(i == pl.num_programs(0) - 1)
        def _():
            logits = jnp.dot(g_acc[...], wf_ref[...],
                             preferred_element_type=jnp.float32) + bf_ref[...]
            m = jnp.max(logits, axis=1, keepdims=True)
            lse = jnp.log(jnp.sum(jnp.exp(logits - m), axis=1, keepdims=True)) + m
            out_ref[...] = logits - lse

    return pl.pallas_call(
        body,
        grid=(n // _BN,),
        in_specs=[
            pl.BlockSpec((_BN, d_out), lambda i: (i, 0)),
            pl.BlockSpec((2, 1, _BN, _D), lambda i: (0, 0, i, 0)),
            pl.BlockSpec((_BN, 1), lambda i: (i, 0)),
            pl.BlockSpec((_BN, 1), lambda i: (i, 0)),
            pl.BlockSpec((d_out, n_cls), lambda i: (0, 0)),
            pl.BlockSpec((1, n_cls), lambda i: (0, 0)),
        ],
        out_specs=pl.BlockSpec((nb, n_cls), lambda i: (0, 0)),
        out_shape=jax.ShapeDtypeStruct((nb, n_cls), jnp.float32),
        scratch_shapes=[pltpu.VMEM((nb, d_out), jnp.float32)],
    )(h20, p2, dis, batch2d, wf, bf)


def kernel(x, edge_attr, W1_0, W1_1, b1, W2_0, W2_1, b2, Wf, bf,
           edge_index, batch, y):
    n = x.shape[0]
    nb = y.shape[0]
    row = edge_index[0]
    col = edge_index[1]

    degp = _sc_deg(row, edge_attr)
    h0, dis, u1a, u1b = _tc12(x, W1_0, W1_1, b1.reshape(1, -1),
                              degp[:, :n].T)
    p1 = _sc_edge_multi((u1a, u1b), row, col, edge_attr)
    h20, u2 = _tc3(h0, p1, dis, W2_0, W2_1, b2.reshape(1, -1))
    p2 = _sc_edge_multi((u2,), row, col, edge_attr)
    out = _tc4(h20, p2, dis, batch.reshape(-1, 1), Wf,
               bf.reshape(1, -1), nb)
    return out
